# baseline (speedup 1.0000x reference)
import functools

import jax
import jax.numpy as jnp
from jax import lax
from jax.experimental import pallas as pl
from jax.experimental.pallas import tpu as pltpu
from jax.experimental.pallas import tpu_sc as plsc

N_NODES = 10000
N_EDGES = 320000
D = 128
BOND = 16

NC = 2
NS = 16
NW = NC * NS
EPW = N_EDGES // NW
CHUNK = 40
NCHUNK = EPW // CHUNK
IB = 25
NBLK = NCHUNK // IB
NPAD = 10240
RPT = NPAD // NS


def _prelude_body(e_ref, w_ref, b_ref, x_ref, ew_ref, xs_ref, xd_ref):
    ew_ref[...] = (
        jnp.dot(e_ref[...], w_ref[2 * D:, :], preferred_element_type=jnp.float32)
        + b_ref[...]
    )

    @pl.when(pl.program_id(0) == 0)
    def _():
        xv = x_ref[...]
        xs_ref[...] = jnp.dot(xv, w_ref[0:D, :],
                              preferred_element_type=jnp.float32)
        xd_ref[...] = jnp.dot(xv, w_ref[D:2 * D, :],
                              preferred_element_type=jnp.float32)


def _sc_edge_body(xs_hbm, xd_hbm, ew_hbm, ei_hbm, out_hbm,
                  i0s, i0d, i1s, i1d,
                  a0, b0, c0, a1, b1, c1,
                  agg_sh,
                  sa0, sb0, sc0, sa1, sb1, sc1):
    cid = lax.axis_index("c")
    sid = lax.axis_index("s")
    wid = sid * NC + cid
    ebase = wid * NCHUNK

    z = jnp.zeros((16,), jnp.float32)

    def zvec(r, carry):
        for k in range(8):
            c0[r, pl.ds(k * 16, 16)] = z
        return carry
    lax.fori_loop(0, CHUNK, zvec, 0)

    def zcopy(j, carry):
        pltpu.sync_copy(c0, agg_sh.at[pl.ds(sid * RPT + j * CHUNK, CHUNK)])
        return carry
    lax.fori_loop(0, RPT // CHUNK, zcopy, 0)
    plsc.subcore_barrier()

    sets = ((a0, b0, c0, sa0, sb0, sc0), (a1, b1, c1, sa1, sb1, sc1))
    iblocks = ((i0s, i0d), (i1s, i1d))

    def fetch_block(b, p):
        ibs, ibd = iblocks[p]
        pltpu.sync_copy(ei_hbm.at[0, wid, b], ibs)
        pltpu.sync_copy(ei_hbm.at[1, wid, b], ibd)

    def start1(jj, s, p):
        a, b, c, sa, sb, sc = sets[s]
        ibs, ibd = iblocks[p]
        off = jj % IB
        pltpu.make_async_copy(xs_hbm.at[ibs.at[off]], a, sa).start()
        pltpu.make_async_copy(xd_hbm.at[ibd.at[off]], b, sb).start()
        pltpu.make_async_copy(
            ew_hbm.at[pl.ds((ebase + jj) * CHUNK, CHUNK)], c, sc).start()

    def start(jj, s):
        par = (jj // IB) % 2

        @pl.when(par == 0)
        def _():
            start1(jj, s, 0)

        @pl.when(par == 1)
        def _():
            start1(jj, s, 1)

    def finish1(jj, s, p):
        a, b, c, sa, sb, sc = sets[s]
        ibs, ibd = iblocks[p]
        off = jj % IB
        pltpu.make_async_copy(xs_hbm.at[ibs.at[off]], a, sa).wait()
        pltpu.make_async_copy(xd_hbm.at[ibd.at[off]], b, sb).wait()
        pltpu.make_async_copy(
            ew_hbm.at[pl.ds((ebase + jj) * CHUNK, CHUNK)], c, sc).wait()

        def rowfn(r, carry):
            for k in range(8):
                sl = pl.ds(k * 16, 16)
                c[r, sl] = jnp.maximum(a[r, sl] + b[r, sl] + c[r, sl], 0.0)
            return carry
        lax.fori_loop(0, CHUNK, rowfn, 0)
        pltpu.sync_copy(c, agg_sh.at[ibd.at[off]], add=True)

    def finish(jj, s):
        par = (jj // IB) % 2

        @pl.when(par == 0)
        def _():
            finish1(jj, s, 0)

        @pl.when(par == 1)
        def _():
            finish1(jj, s, 1)

    fetch_block(0, 0)
    start(0, 0)

    def body(j, carry):
        @pl.when(j % IB == 0)
        def _():
            nb = j // IB + 1

            @pl.when(jnp.logical_and(nb < NBLK, nb % 2 == 0))
            def _():
                fetch_block(nb, 0)

            @pl.when(jnp.logical_and(nb < NBLK, nb % 2 == 1))
            def _():
                fetch_block(nb, 1)

        nxt = j + 1
        even = (j % 2) == 0

        @pl.when(jnp.logical_and(nxt < NCHUNK, jnp.logical_not(even)))
        def _():
            start(nxt, 0)

        @pl.when(jnp.logical_and(nxt < NCHUNK, even))
        def _():
            start(nxt, 1)

        @pl.when(even)
        def _():
            finish(j, 0)

        @pl.when(jnp.logical_not(even))
        def _():
            finish(j, 1)
        return carry
    lax.fori_loop(0, NCHUNK, body, 0)

    plsc.subcore_barrier()
    pltpu.sync_copy(agg_sh.at[pl.ds(sid * RPT, RPT)],
                    out_hbm.at[cid, pl.ds(sid * RPT, RPT)])


def _update_body(x_ref, agg_ref, w_ref, b_ref, h_ref, p_ref):
    agg = (agg_ref[0] + agg_ref[1])[0:N_NODES, :]
    u = (jnp.dot(x_ref[...], w_ref[0:D, :], preferred_element_type=jnp.float32)
         + jnp.dot(agg, w_ref[D:, :], preferred_element_type=jnp.float32)
         + b_ref[...])
    h = jnp.maximum(u, 0.0)
    h_ref[...] = h
    p_ref[...] = jnp.sum(h, axis=0, keepdims=True)


def kernel(x, edge_index, edge_attr, W_msg, b_msg, W_upd, b_upd):
    ei = edge_index.astype(jnp.int32).reshape(2, NW, NBLK, IB, CHUNK)

    BE = 4000
    ew, xs, xd = pl.pallas_call(
        _prelude_body,
        grid=(N_EDGES // BE,),
        in_specs=[
            pl.BlockSpec((BE, BOND), lambda i: (i, 0)),
            pl.BlockSpec((2 * D + BOND, D), lambda i: (0, 0)),
            pl.BlockSpec((1, D), lambda i: (0, 0)),
            pl.BlockSpec((N_NODES, D), lambda i: (0, 0)),
        ],
        out_specs=(
            pl.BlockSpec((BE, D), lambda i: (i, 0)),
            pl.BlockSpec((N_NODES, D), lambda i: (0, 0)),
            pl.BlockSpec((N_NODES, D), lambda i: (0, 0)),
        ),
        out_shape=(
            jax.ShapeDtypeStruct((N_EDGES, D), jnp.float32),
            jax.ShapeDtypeStruct((N_NODES, D), jnp.float32),
            jax.ShapeDtypeStruct((N_NODES, D), jnp.float32),
        ),
    )(edge_attr, W_msg, b_msg.reshape(1, D), x)

    sc_edge = functools.partial(
        pl.kernel,
        out_type=jax.ShapeDtypeStruct((NC, NPAD, D), jnp.float32),
        mesh=plsc.VectorSubcoreMesh(core_axis_name="c", subcore_axis_name="s"),
        scratch_types=(
            [pltpu.VMEM((IB, CHUNK), jnp.int32)] * 4
            + [pltpu.VMEM((CHUNK, D), jnp.float32)] * 6
            + [pltpu.VMEM_SHARED((NPAD, D), jnp.float32)]
            + [pltpu.SemaphoreType.DMA] * 6
        ),
    )(_sc_edge_body)
    agg2 = sc_edge(xs, xd, ew, ei)

    h, pooled = pl.pallas_call(
        _update_body,
        out_shape=(
            jax.ShapeDtypeStruct((N_NODES, D), jnp.float32),
            jax.ShapeDtypeStruct((1, D), jnp.float32),
        ),
    )(x, agg2, W_upd, b_upd.reshape(1, D))
    return (h, pooled)


# speedup vs baseline: 1.0654x; 1.0654x over previous
import functools

import jax
import jax.numpy as jnp
from jax import lax
from jax.experimental import pallas as pl
from jax.experimental.pallas import tpu as pltpu
from jax.experimental.pallas import tpu_sc as plsc

N_NODES = 10000
N_EDGES = 320000
D = 128
BOND = 16

NC = 2
NS = 16
NW = NC * NS
EPW = N_EDGES // NW
CHUNK = 40
NCHUNK = EPW // CHUNK
IB = 25
NBLK = NCHUNK // IB
NPAD = 10240
RPT = NPAD // NS
BE = EPW
BE2 = BE // 2
D2 = D // 2
NSUPER = BE2 // CHUNK


def _prelude_body(e_ref, w_ref, b_ref, x_ref, ew_ref, xs_ref, xd_ref):
    ew = (
        jnp.dot(e_ref[...], w_ref[2 * D:, :], preferred_element_type=jnp.float32)
        + b_ref[...]
    )
    bits = jax.lax.bitcast_convert_type(ew, jnp.uint32)
    rnd = bits + jnp.uint32(0x7FFF) + ((bits >> jnp.uint32(16)) & jnp.uint32(1))
    bf = rnd & jnp.uint32(0xFFFF0000)
    word = (bf[:, 0:D2] >> jnp.uint32(16)) | bf[:, D2:D]
    ew_ref[...] = jax.lax.bitcast_convert_type(
        jnp.concatenate([word[0:BE2, :], word[BE2:BE, :]], axis=1), jnp.int32)

    @pl.when(pl.program_id(0) == 0)
    def _():
        xv = x_ref[...]
        xs_ref[...] = jnp.dot(xv, w_ref[0:D, :],
                              preferred_element_type=jnp.float32)
        xd_ref[...] = jnp.dot(xv, w_ref[D:2 * D, :],
                              preferred_element_type=jnp.float32)


def _sc_edge_body(xs_hbm, xd_hbm, ew_hbm, ei_hbm, out_hbm,
                  i0s, i0d, i1s, i1d,
                  a0, b0, a1, b1, e0, e1,
                  agg_sh,
                  sa0, sb0, sa1, sb1, se0, se1):
    cid = lax.axis_index("c")
    sid = lax.axis_index("s")
    wid = sid * NC + cid
    rowbase = wid * BE2

    z = jnp.zeros((16,), jnp.float32)

    def zvec(r, carry):
        for k in range(8):
            a0[r, pl.ds(k * 16, 16)] = z
        return carry
    lax.fori_loop(0, CHUNK, zvec, 0)

    def zcopy(j, carry):
        pltpu.sync_copy(a0, agg_sh.at[pl.ds(sid * RPT + j * CHUNK, CHUNK)])
        return carry
    lax.fori_loop(0, RPT // CHUNK, zcopy, 0)
    plsc.subcore_barrier()

    gsets = ((a0, b0, sa0, sb0), (a1, b1, sa1, sb1))
    ebufs = ((e0, se0), (e1, se1))
    iblocks = ((i0s, i0d), (i1s, i1d))

    def fetch_block(b, p):
        ibs, ibd = iblocks[p]
        pltpu.sync_copy(ei_hbm.at[0, wid, b], ibs)
        pltpu.sync_copy(ei_hbm.at[1, wid, b], ibd)

    def ew_desc(j, p):
        e, se = ebufs[p]
        return pltpu.make_async_copy(
            ew_hbm.at[pl.ds(rowbase + j * CHUNK, CHUNK)], e, se)

    def start1(t, s, p):
        a, b, sa, sb = gsets[s]
        ibs, ibd = iblocks[p]
        off = t % IB
        pltpu.make_async_copy(xs_hbm.at[ibs.at[off]], a, sa).start()
        pltpu.make_async_copy(xd_hbm.at[ibd.at[off]], b, sb).start()

    def start(t, s):
        par = (t // IB) % 2

        @pl.when(par == 0)
        def _():
            start1(t, s, 0)

        @pl.when(par == 1)
        def _():
            start1(t, s, 1)

    def compute(s, ep):
        a, b, sa, sb = gsets[s]
        e, se = ebufs[ep]

        def rowfn(r, carry):
            for g in range(4):
                w16 = e[r, pl.ds(D2 * s + 16 * g, 16)]
                lo = jax.lax.bitcast_convert_type(
                    jnp.left_shift(w16, 16), jnp.float32)
                hi = jax.lax.bitcast_convert_type(
                    jnp.bitwise_and(w16, jnp.int32(-65536)), jnp.float32)
                sll = pl.ds(16 * g, 16)
                slh = pl.ds(D2 + 16 * g, 16)
                a[r, sll] = jnp.maximum(a[r, sll] + b[r, sll] + lo, 0.0)
                a[r, slh] = jnp.maximum(a[r, slh] + b[r, slh] + hi, 0.0)
            return carry
        lax.fori_loop(0, CHUNK, rowfn, 0)

    def finish1(t, s, p):
        a, b, sa, sb = gsets[s]
        ibs, ibd = iblocks[p]
        off = t % IB
        pltpu.make_async_copy(xs_hbm.at[ibs.at[off]], a, sa).wait()
        pltpu.make_async_copy(xd_hbm.at[ibd.at[off]], b, sb).wait()
        j = t // 2

        @pl.when(j % 2 == 0)
        def _():
            if s == 0:
                ew_desc(j, 0).wait()
            compute(s, 0)

        @pl.when(j % 2 == 1)
        def _():
            if s == 0:
                ew_desc(j, 1).wait()
            compute(s, 1)

        pltpu.sync_copy(a, agg_sh.at[ibd.at[off]], add=True)

    def finish(t, s):
        par = (t // IB) % 2

        @pl.when(par == 0)
        def _():
            finish1(t, s, 0)

        @pl.when(par == 1)
        def _():
            finish1(t, s, 1)

    fetch_block(0, 0)
    ew_desc(0, 0).start()
    start(0, 0)

    def body(t, carry):
        @pl.when(t % IB == 0)
        def _():
            nb = t // IB + 1

            @pl.when(jnp.logical_and(nb < NBLK, nb % 2 == 0))
            def _():
                fetch_block(nb, 0)

            @pl.when(jnp.logical_and(nb < NBLK, nb % 2 == 1))
            def _():
                fetch_block(nb, 1)

        @pl.when(t % 2 == 0)
        def _():
            nj = t // 2 + 1

            @pl.when(jnp.logical_and(nj < NSUPER, nj % 2 == 0))
            def _():
                ew_desc(nj, 0).start()

            @pl.when(jnp.logical_and(nj < NSUPER, nj % 2 == 1))
            def _():
                ew_desc(nj, 1).start()

        nxt = t + 1
        even = (t % 2) == 0

        @pl.when(jnp.logical_and(nxt < NCHUNK, jnp.logical_not(even)))
        def _():
            start(nxt, 0)

        @pl.when(jnp.logical_and(nxt < NCHUNK, even))
        def _():
            start(nxt, 1)

        @pl.when(even)
        def _():
            finish(t, 0)

        @pl.when(jnp.logical_not(even))
        def _():
            finish(t, 1)
        return carry
    lax.fori_loop(0, NCHUNK, body, 0)

    plsc.subcore_barrier()
    pltpu.sync_copy(agg_sh.at[pl.ds(sid * RPT, RPT)],
                    out_hbm.at[cid, pl.ds(sid * RPT, RPT)])


def _update_body(x_ref, agg_ref, w_ref, b_ref, h_ref, p_ref):
    agg = (agg_ref[0] + agg_ref[1])[0:N_NODES, :]
    u = (jnp.dot(x_ref[...], w_ref[0:D, :], preferred_element_type=jnp.float32)
         + jnp.dot(agg, w_ref[D:, :], preferred_element_type=jnp.float32)
         + b_ref[...])
    h = jnp.maximum(u, 0.0)
    h_ref[...] = h
    p_ref[...] = jnp.sum(h, axis=0, keepdims=True)


def kernel(x, edge_index, edge_attr, W_msg, b_msg, W_upd, b_upd):
    ei = (edge_index.astype(jnp.int32)
          .reshape(2, NW, 2, NSUPER, CHUNK)
          .transpose(0, 1, 3, 2, 4)
          .reshape(2, NW, NBLK, IB, CHUNK))

    ew, xs, xd = pl.pallas_call(
        _prelude_body,
        grid=(N_EDGES // BE,),
        in_specs=[
            pl.BlockSpec((BE, BOND), lambda i: (i, 0)),
            pl.BlockSpec((2 * D + BOND, D), lambda i: (0, 0)),
            pl.BlockSpec((1, D), lambda i: (0, 0)),
            pl.BlockSpec((N_NODES, D), lambda i: (0, 0)),
        ],
        out_specs=(
            pl.BlockSpec((BE2, D), lambda i: (i, 0)),
            pl.BlockSpec((N_NODES, D), lambda i: (0, 0)),
            pl.BlockSpec((N_NODES, D), lambda i: (0, 0)),
        ),
        out_shape=(
            jax.ShapeDtypeStruct((N_EDGES // 2, D), jnp.int32),
            jax.ShapeDtypeStruct((N_NODES, D), jnp.float32),
            jax.ShapeDtypeStruct((N_NODES, D), jnp.float32),
        ),
    )(edge_attr, W_msg, b_msg.reshape(1, D), x)

    sc_edge = functools.partial(
        pl.kernel,
        out_type=jax.ShapeDtypeStruct((NC, NPAD, D), jnp.float32),
        mesh=plsc.VectorSubcoreMesh(core_axis_name="c", subcore_axis_name="s"),
        scratch_types=(
            [pltpu.VMEM((IB, CHUNK), jnp.int32)] * 4
            + [pltpu.VMEM((CHUNK, D), jnp.float32)] * 4
            + [pltpu.VMEM((CHUNK, D), jnp.int32)] * 2
            + [pltpu.VMEM_SHARED((NPAD, D), jnp.float32)]
            + [pltpu.SemaphoreType.DMA] * 6
        ),
    )(_sc_edge_body)
    agg2 = sc_edge(xs, xd, ew, ei)

    h, pooled = pl.pallas_call(
        _update_body,
        out_shape=(
            jax.ShapeDtypeStruct((N_NODES, D), jnp.float32),
            jax.ShapeDtypeStruct((1, D), jnp.float32),
        ),
    )(x, agg2, W_upd, b_upd.reshape(1, D))
    return (h, pooled)


# speedup vs baseline: 1.0770x; 1.0110x over previous
import functools

import jax
import jax.numpy as jnp
from jax import lax
from jax.experimental import pallas as pl
from jax.experimental.pallas import tpu as pltpu
from jax.experimental.pallas import tpu_sc as plsc

N_NODES = 10000
N_EDGES = 320000
D = 128
BOND = 16

NC = 2
NS = 16
NW = NC * NS
EPW = N_EDGES // NW
CHUNK = 40
NCHUNK = EPW // CHUNK
IB = 25
NBLK = NCHUNK // IB
NPAD = 10240
RPT = NPAD // NS
BE = EPW
BE2 = BE // 2
D2 = D // 2
NSUPER = BE2 // CHUNK


def _prelude_body(e_ref, w_ref, b_ref, x_ref, ew_ref, xs_ref, xd_ref):
    ew = (
        jnp.dot(e_ref[...].astype(jnp.bfloat16),
                w_ref[2 * D:, :].astype(jnp.bfloat16),
                preferred_element_type=jnp.float32)
        + b_ref[...]
    )
    bits = jax.lax.bitcast_convert_type(ew, jnp.uint32)
    bf = bits & jnp.uint32(0xFFFF0000)
    word = (bf[:, 0:D2] >> jnp.uint32(16)) | bf[:, D2:D]
    ew_ref[...] = jax.lax.bitcast_convert_type(
        jnp.concatenate([word[0:BE2, :], word[BE2:BE, :]], axis=1), jnp.int32)

    @pl.when(pl.program_id(0) == 0)
    def _():
        xv = x_ref[...]
        xs_ref[...] = jnp.dot(xv, w_ref[0:D, :],
                              preferred_element_type=jnp.float32)
        xd_ref[...] = jnp.dot(xv, w_ref[D:2 * D, :],
                              preferred_element_type=jnp.float32)


def _sc_edge_body(xs_hbm, xd_hbm, ew_hbm, ei_hbm, out_hbm,
                  i0s, i0d, i1s, i1d,
                  a0, b0, a1, b1, e0, e1,
                  agg_sh,
                  sa0, sb0, sa1, sb1, se0, se1):
    cid = lax.axis_index("c")
    sid = lax.axis_index("s")
    wid = sid * NC + cid
    rowbase = wid * BE2

    z = jnp.zeros((16,), jnp.float32)

    def zvec(r, carry):
        for k in range(8):
            a0[r, pl.ds(k * 16, 16)] = z
        return carry
    lax.fori_loop(0, CHUNK, zvec, 0)

    def zcopy(j, carry):
        pltpu.sync_copy(a0, agg_sh.at[pl.ds(sid * RPT + j * CHUNK, CHUNK)])
        return carry
    lax.fori_loop(0, RPT // CHUNK, zcopy, 0)
    plsc.subcore_barrier()

    gsets = ((a0, b0, sa0, sb0), (a1, b1, sa1, sb1))
    ebufs = ((e0, se0), (e1, se1))
    iblocks = ((i0s, i0d), (i1s, i1d))

    def fetch_block(b, p):
        ibs, ibd = iblocks[p]
        pltpu.sync_copy(ei_hbm.at[0, wid, b], ibs)
        pltpu.sync_copy(ei_hbm.at[1, wid, b], ibd)

    def ew_desc(j, p):
        e, se = ebufs[p]
        return pltpu.make_async_copy(
            ew_hbm.at[pl.ds(rowbase + j * CHUNK, CHUNK)], e, se)

    def start1(t, s, p):
        a, b, sa, sb = gsets[s]
        ibs, ibd = iblocks[p]
        off = t % IB
        pltpu.make_async_copy(xs_hbm.at[ibs.at[off]], a, sa).start()
        pltpu.make_async_copy(xd_hbm.at[ibd.at[off]], b, sb).start()

    def start(t, s):
        par = (t // IB) % 2

        @pl.when(par == 0)
        def _():
            start1(t, s, 0)

        @pl.when(par == 1)
        def _():
            start1(t, s, 1)

    def compute(s, ep):
        a, b, sa, sb = gsets[s]
        e, se = ebufs[ep]

        def rowfn(r, carry):
            for g in range(4):
                w16 = e[r, pl.ds(D2 * s + 16 * g, 16)]
                lo = jax.lax.bitcast_convert_type(
                    jnp.left_shift(w16, 16), jnp.float32)
                hi = jax.lax.bitcast_convert_type(
                    jnp.bitwise_and(w16, jnp.int32(-65536)), jnp.float32)
                sll = pl.ds(16 * g, 16)
                slh = pl.ds(D2 + 16 * g, 16)
                a[r, sll] = jnp.maximum(a[r, sll] + b[r, sll] + lo, 0.0)
                a[r, slh] = jnp.maximum(a[r, slh] + b[r, slh] + hi, 0.0)
            return carry
        lax.fori_loop(0, CHUNK, rowfn, 0)

    def finish1(t, s, p):
        a, b, sa, sb = gsets[s]
        ibs, ibd = iblocks[p]
        off = t % IB
        pltpu.make_async_copy(xs_hbm.at[ibs.at[off]], a, sa).wait()
        pltpu.make_async_copy(xd_hbm.at[ibd.at[off]], b, sb).wait()
        j = t // 2

        @pl.when(j % 2 == 0)
        def _():
            if s == 0:
                ew_desc(j, 0).wait()
            compute(s, 0)

        @pl.when(j % 2 == 1)
        def _():
            if s == 0:
                ew_desc(j, 1).wait()
            compute(s, 1)

        pltpu.sync_copy(a, agg_sh.at[ibd.at[off]], add=True)

    def finish(t, s):
        par = (t // IB) % 2

        @pl.when(par == 0)
        def _():
            finish1(t, s, 0)

        @pl.when(par == 1)
        def _():
            finish1(t, s, 1)

    fetch_block(0, 0)
    ew_desc(0, 0).start()
    start(0, 0)

    def body(t, carry):
        @pl.when(t % IB == 0)
        def _():
            nb = t // IB + 1

            @pl.when(jnp.logical_and(nb < NBLK, nb % 2 == 0))
            def _():
                fetch_block(nb, 0)

            @pl.when(jnp.logical_and(nb < NBLK, nb % 2 == 1))
            def _():
                fetch_block(nb, 1)

        @pl.when(t % 2 == 0)
        def _():
            nj = t // 2 + 1

            @pl.when(jnp.logical_and(nj < NSUPER, nj % 2 == 0))
            def _():
                ew_desc(nj, 0).start()

            @pl.when(jnp.logical_and(nj < NSUPER, nj % 2 == 1))
            def _():
                ew_desc(nj, 1).start()

        nxt = t + 1
        even = (t % 2) == 0

        @pl.when(jnp.logical_and(nxt < NCHUNK, jnp.logical_not(even)))
        def _():
            start(nxt, 0)

        @pl.when(jnp.logical_and(nxt < NCHUNK, even))
        def _():
            start(nxt, 1)

        @pl.when(even)
        def _():
            finish(t, 0)

        @pl.when(jnp.logical_not(even))
        def _():
            finish(t, 1)
        return carry
    lax.fori_loop(0, NCHUNK, body, 0)

    plsc.subcore_barrier()
    pltpu.sync_copy(agg_sh.at[pl.ds(sid * RPT, RPT)],
                    out_hbm.at[cid, pl.ds(sid * RPT, RPT)])


def _update_body(x_ref, agg_ref, w_ref, b_ref, h_ref, p_ref):
    agg = (agg_ref[0] + agg_ref[1])[0:N_NODES, :]
    u = (jnp.dot(x_ref[...], w_ref[0:D, :], preferred_element_type=jnp.float32)
         + jnp.dot(agg, w_ref[D:, :], preferred_element_type=jnp.float32)
         + b_ref[...])
    h = jnp.maximum(u, 0.0)
    h_ref[...] = h
    p_ref[...] = jnp.sum(h, axis=0, keepdims=True)


def kernel(x, edge_index, edge_attr, W_msg, b_msg, W_upd, b_upd):
    ei = (edge_index.astype(jnp.int32)
          .reshape(2, NW, 2, NSUPER, CHUNK)
          .transpose(0, 1, 3, 2, 4)
          .reshape(2, NW, NBLK, IB, CHUNK))

    ew, xs, xd = pl.pallas_call(
        _prelude_body,
        grid=(N_EDGES // BE,),
        in_specs=[
            pl.BlockSpec((BE, BOND), lambda i: (i, 0)),
            pl.BlockSpec((2 * D + BOND, D), lambda i: (0, 0)),
            pl.BlockSpec((1, D), lambda i: (0, 0)),
            pl.BlockSpec((N_NODES, D), lambda i: (0, 0)),
        ],
        out_specs=(
            pl.BlockSpec((BE2, D), lambda i: (i, 0)),
            pl.BlockSpec((N_NODES, D), lambda i: (0, 0)),
            pl.BlockSpec((N_NODES, D), lambda i: (0, 0)),
        ),
        out_shape=(
            jax.ShapeDtypeStruct((N_EDGES // 2, D), jnp.int32),
            jax.ShapeDtypeStruct((N_NODES, D), jnp.float32),
            jax.ShapeDtypeStruct((N_NODES, D), jnp.float32),
        ),
    )(edge_attr, W_msg, b_msg.reshape(1, D), x)

    sc_edge = functools.partial(
        pl.kernel,
        out_type=jax.ShapeDtypeStruct((NC, NPAD, D), jnp.float32),
        mesh=plsc.VectorSubcoreMesh(core_axis_name="c", subcore_axis_name="s"),
        scratch_types=(
            [pltpu.VMEM((IB, CHUNK), jnp.int32)] * 4
            + [pltpu.VMEM((CHUNK, D), jnp.float32)] * 4
            + [pltpu.VMEM((CHUNK, D), jnp.int32)] * 2
            + [pltpu.VMEM_SHARED((NPAD, D), jnp.float32)]
            + [pltpu.SemaphoreType.DMA] * 6
        ),
    )(_sc_edge_body)
    agg2 = sc_edge(xs, xd, ew, ei)

    h, pooled = pl.pallas_call(
        _update_body,
        out_shape=(
            jax.ShapeDtypeStruct((N_NODES, D), jnp.float32),
            jax.ShapeDtypeStruct((1, D), jnp.float32),
        ),
    )(x, agg2, W_upd, b_upd.reshape(1, D))
    return (h, pooled)
